# Initial kernel scaffold; baseline (speedup 1.0000x reference)
import jax
import jax.numpy as jnp
from jax.experimental import pallas as pl


def kernel(x, edge_index, W1, b1, W2, b2):
    raise NotImplementedError("write your pallas kernel here")


# speedup vs baseline: 17.3179x; 17.3179x over previous
import functools

import jax
import jax.numpy as jnp
from jax import lax
from jax.experimental import pallas as pl
from jax.experimental.pallas import tpu as pltpu
from jax.experimental.pallas import tpu_sc as plsc

N = 10000
NP = 10240
E = 320000
IN_CH = 128
HID = 16
NCL = 40
W = 128
NC = 2
NS = 16
NW = NC * NS
EPW = E // NW
K = 80
NCH = EPW // K
RPT = NP // NS

_mesh = plsc.VectorSubcoreMesh(core_axis_name="c", subcore_axis_name="s")


def _sc_degree(dst3, ones, zeros):
    @functools.partial(
        pl.kernel,
        mesh=_mesh,
        out_type=jax.ShapeDtypeStruct((NC, NP, 1, W), jnp.float32),
        scratch_types=[
            pltpu.VMEM((K, 1, W), jnp.float32),
            pltpu.VMEM((NCH, K), jnp.int32),
            pltpu.VMEM_SHARED((NP, 1, W), jnp.float32),
        ],
    )
    def k(dst_hbm, ones_hbm, zeros_hbm, out_hbm, ones_v, idx_v, deg_sh):
        c = lax.axis_index("c")
        s = lax.axis_index("s")
        w = c * NS + s
        base = s * RPT
        pltpu.sync_copy(ones_hbm, ones_v)
        pltpu.sync_copy(zeros_hbm.at[pl.ds(base, RPT)],
                        deg_sh.at[pl.ds(base, RPT)])
        pltpu.sync_copy(dst_hbm.at[w], idx_v)
        plsc.subcore_barrier()

        @pl.loop(0, NCH)
        def _(j):
            pltpu.sync_copy(ones_v, deg_sh.at[idx_v.at[j]], add=True)

        plsc.subcore_barrier()
        pltpu.sync_copy(deg_sh.at[pl.ds(base, RPT)],
                        out_hbm.at[c, pl.ds(base, RPT)])

    return k(dst3, ones, zeros)


def _sc_scatter(g, src3, dst3, zeros):
    @functools.partial(
        pl.kernel,
        mesh=_mesh,
        out_type=jax.ShapeDtypeStruct((NC, NP, 1, W), jnp.float32),
        scratch_types=[
            pltpu.VMEM((NCH, K), jnp.int32),
            pltpu.VMEM((NCH, K), jnp.int32),
            pltpu.VMEM((K, 1, W), jnp.float32),
            pltpu.VMEM_SHARED((NP, 1, W), jnp.float32),
            pltpu.SemaphoreType.DMA,
        ],
    )
    def k(g_hbm, src_hbm, dst_hbm, zeros_hbm, out_hbm,
          src_v, dst_v, rows_v, acc_sh, sem):
        c = lax.axis_index("c")
        s = lax.axis_index("s")
        w = c * NS + s
        base = s * RPT
        pltpu.sync_copy(zeros_hbm.at[pl.ds(base, RPT)],
                        acc_sh.at[pl.ds(base, RPT)])
        pltpu.sync_copy(src_hbm.at[w], src_v)
        pltpu.sync_copy(dst_hbm.at[w], dst_v)
        plsc.subcore_barrier()

        @pl.loop(0, NCH)
        def _(j):
            pltpu.async_copy(g_hbm.at[src_v.at[j]], rows_v, sem).wait()
            pltpu.sync_copy(rows_v, acc_sh.at[dst_v.at[j]], add=True)

        plsc.subcore_barrier()
        pltpu.sync_copy(acc_sh.at[pl.ds(base, RPT)],
                        out_hbm.at[c, pl.ds(base, RPT)])

    return k(g, src3, dst3, zeros)


def _tc_matmul1(x, W1):
    def body(x_ref, w_ref, o_ref):
        o_ref[...] = jnp.dot(x_ref[...], w_ref[...],
                             preferred_element_type=jnp.float32)

    return pl.pallas_call(
        body, out_shape=jax.ShapeDtypeStruct((N, HID), jnp.float32))(x, W1)


def _tc_norm(degp, h1):
    def body(dp_ref, h1_ref, dinv_ref, g_ref):
        deg = dp_ref[0, :N, :HID] + dp_ref[1, :N, :HID] + 1.0
        dinv = lax.rsqrt(deg)
        dinv_ref[...] = dinv
        g_ref[:N, :HID] = dinv * h1_ref[...]
        g_ref[:N, HID:] = jnp.zeros((N, W - HID), jnp.float32)
        g_ref[N:, :] = jnp.zeros((NP - N, W), jnp.float32)

    return pl.pallas_call(
        body,
        out_shape=[jax.ShapeDtypeStruct((N, HID), jnp.float32),
                   jax.ShapeDtypeStruct((NP, W), jnp.float32)],
    )(degp, h1)


def _tc_layer1(parts, g, dinv16, b1):
    def body(p_ref, g_ref, d_ref, b_ref, g2_ref):
        dinv = d_ref[...]
        h = dinv * (p_ref[0, :N, :HID] + p_ref[1, :N, :HID]
                    + g_ref[:N, :HID]) + b_ref[...][None, :]
        h = jnp.maximum(h, 0.0)
        g2_ref[:N, :HID] = dinv * h
        g2_ref[:N, HID:] = jnp.zeros((N, W - HID), jnp.float32)
        g2_ref[N:, :] = jnp.zeros((NP - N, W), jnp.float32)

    return pl.pallas_call(
        body,
        out_shape=jax.ShapeDtypeStruct((NP, W), jnp.float32),
    )(parts, g, dinv16, b1)


def _tc_layer2(parts, g2, dinv16, W2, b2):
    def body(p_ref, g2_ref, d_ref, w_ref, b_ref, o_ref):
        m = d_ref[...] * (p_ref[0, :N, :HID] + p_ref[1, :N, :HID]
                          + g2_ref[:N, :HID])
        o_ref[...] = jnp.dot(m, w_ref[...],
                             preferred_element_type=jnp.float32) \
            + b_ref[...][None, :]

    return pl.pallas_call(
        body,
        out_shape=jax.ShapeDtypeStruct((N, NCL), jnp.float32),
    )(parts, g2, dinv16, W2, b2)


def kernel(x, edge_index, W1, b1, W2, b2):
    idx = edge_index.astype(jnp.int32)
    src3 = idx[0].reshape(NW, NCH, K)
    dst3 = idx[1].reshape(NW, NCH, K)

    ones = jnp.ones((K, 1, W), jnp.float32)
    zeros = jnp.zeros((NP, 1, W), jnp.float32)
    degp = _sc_degree(dst3, ones, zeros)
    h1 = _tc_matmul1(x, W1)
    dinv16, g = _tc_norm(degp.reshape(NC, NP, W), h1)
    parts1 = _sc_scatter(g.reshape(NP, 1, W), src3, dst3, zeros)
    g2 = _tc_layer1(parts1.reshape(NC, NP, W), g, dinv16, b1)
    parts2 = _sc_scatter(g2.reshape(NP, 1, W), src3, dst3, zeros)
    return _tc_layer2(parts2.reshape(NC, NP, W), g2, dinv16, W2, b2)


# speedup vs baseline: 18.4202x; 1.0637x over previous
import functools

import jax
import jax.numpy as jnp
from jax import lax
from jax.experimental import pallas as pl
from jax.experimental.pallas import tpu as pltpu
from jax.experimental.pallas import tpu_sc as plsc

N = 10000
NP = 10240
E = 320000
IN_CH = 128
HID = 16
NCL = 40
W = 128
NC = 2
NS = 16
NW = NC * NS
EPW = E // NW
K = 80
PAD = 240
EPWP = EPW + PAD
NCH = EPWP // K
NB = 2
IB = 8
RPT = NP // NS

_mesh = plsc.VectorSubcoreMesh(core_axis_name="c", subcore_axis_name="s")


def _sc_degree(epack, ones, zeros):
    @functools.partial(
        pl.kernel,
        mesh=_mesh,
        out_type=jax.ShapeDtypeStruct((NC, NP, 1, W), jnp.float32),
        scratch_types=[
            pltpu.VMEM((K, 1, W), jnp.float32),
        ] + [pltpu.VMEM((K,), jnp.int32) for _ in range(IB)] + [
            pltpu.VMEM_SHARED((NP, 1, W), jnp.float32),
        ] + [pltpu.SemaphoreType.DMA for _ in range(2 * IB)],
    )
    def k(e_hbm, ones_hbm, zeros_hbm, out_hbm, ones_v, *rest):
        ibuf = rest[:IB]
        deg_sh = rest[IB]
        isem = rest[IB + 1:IB + 1 + IB]
        ssem = rest[IB + 1 + IB:]
        c = lax.axis_index("c")
        s = lax.axis_index("s")
        w = c * NS + s
        base = s * RPT
        pltpu.sync_copy(ones_hbm, ones_v)
        pltpu.sync_copy(zeros_hbm.at[pl.ds(base, RPT)],
                        deg_sh.at[pl.ds(base, RPT)])
        plsc.subcore_barrier()

        for b in range(IB):
            pltpu.async_copy(e_hbm.at[w, b, 1], ibuf[b], isem[b])

        @pl.loop(0, NCH, step=IB)
        def _(j):
            for b in range(IB):
                jj = j + b
                pltpu.make_async_copy(e_hbm.at[w, jj, 1], ibuf[b],
                                      isem[b]).wait()
                pltpu.async_copy(ones_v, deg_sh.at[ibuf[b]], ssem[b],
                                 add=True)
            for b in range(IB):
                jj = j + b

                @pl.when(jj + IB < NCH)
                def _():
                    pltpu.make_async_copy(ones_v, deg_sh.at[ibuf[b]],
                                          ssem[b]).wait()
                    pltpu.async_copy(e_hbm.at[w, jj + IB, 1], ibuf[b],
                                     isem[b])

        for b in range(IB):
            pltpu.make_async_copy(ones_v, deg_sh.at[ibuf[b]], ssem[b]).wait()

        plsc.subcore_barrier()
        pltpu.sync_copy(deg_sh.at[pl.ds(base, RPT)],
                        out_hbm.at[c, pl.ds(base, RPT)])

    return k(epack, ones, zeros)


def _sc_scatter(g, epack, zeros):
    @functools.partial(
        pl.kernel,
        mesh=_mesh,
        out_type=jax.ShapeDtypeStruct((NC, NP, 1, W), jnp.float32),
        scratch_types=[pltpu.VMEM((2, K), jnp.int32) for _ in range(NB)]
        + [pltpu.VMEM((K, 1, W), jnp.float32) for _ in range(NB)] + [
            pltpu.VMEM_SHARED((NP, 1, W), jnp.float32),
        ] + [pltpu.SemaphoreType.DMA for _ in range(3 * NB)],
    )
    def k(g_hbm, e_hbm, zeros_hbm, out_hbm, *rest):
        ibuf = rest[:NB]
        bufs = rest[NB:2 * NB]
        acc_sh = rest[2 * NB]
        isem = rest[2 * NB + 1:3 * NB + 1]
        gsem = rest[3 * NB + 1:4 * NB + 1]
        ssem = rest[4 * NB + 1:]
        c = lax.axis_index("c")
        s = lax.axis_index("s")
        w = c * NS + s
        base = s * RPT
        pltpu.sync_copy(zeros_hbm.at[pl.ds(base, RPT)],
                        acc_sh.at[pl.ds(base, RPT)])
        plsc.subcore_barrier()

        for b in range(NB):
            pltpu.async_copy(e_hbm.at[w, b], ibuf[b], isem[b])
        for b in range(NB):
            pltpu.make_async_copy(e_hbm.at[w, b], ibuf[b], isem[b]).wait()
            pltpu.async_copy(g_hbm.at[ibuf[b].at[0]], bufs[b], gsem[b])

        @pl.loop(0, NCH, step=NB)
        def _(j):
            for b in range(NB):
                jj = j + b
                pltpu.make_async_copy(g_hbm.at[ibuf[b].at[0]], bufs[b],
                                      gsem[b]).wait()
                pltpu.async_copy(bufs[b], acc_sh.at[ibuf[b].at[1]], ssem[b],
                                 add=True)
            for b in range(NB):
                jj = j + b

                @pl.when(jj + NB < NCH)
                def _():
                    pltpu.make_async_copy(bufs[b], acc_sh.at[ibuf[b].at[1]],
                                          ssem[b]).wait()
                    pltpu.async_copy(e_hbm.at[w, jj + NB], ibuf[b], isem[b])
            for b in range(NB):
                jj = j + b

                @pl.when(jj + NB < NCH)
                def _():
                    pltpu.make_async_copy(e_hbm.at[w, jj + NB], ibuf[b],
                                          isem[b]).wait()
                    pltpu.async_copy(g_hbm.at[ibuf[b].at[0]], bufs[b],
                                     gsem[b])

        for b in range(NB):
            pltpu.make_async_copy(bufs[b], acc_sh.at[ibuf[b].at[1]],
                                  ssem[b]).wait()

        plsc.subcore_barrier()
        pltpu.sync_copy(acc_sh.at[pl.ds(base, RPT)],
                        out_hbm.at[c, pl.ds(base, RPT)])

    return k(g, epack, zeros)


def _tc_matmul1(x, W1):
    def body(x_ref, w_ref, o_ref):
        o_ref[...] = jnp.dot(x_ref[...], w_ref[...],
                             preferred_element_type=jnp.float32)

    return pl.pallas_call(
        body, out_shape=jax.ShapeDtypeStruct((N, HID), jnp.float32))(x, W1)


def _tc_norm(degp, h1):
    def body(dp_ref, h1_ref, dinv_ref, g_ref):
        deg = dp_ref[0, :N, :HID] + dp_ref[1, :N, :HID] + 1.0
        dinv = lax.rsqrt(deg)
        dinv_ref[...] = dinv
        g_ref[:N, :HID] = dinv * h1_ref[...]
        g_ref[:N, HID:] = jnp.zeros((N, W - HID), jnp.float32)
        g_ref[N:, :] = jnp.zeros((NP - N, W), jnp.float32)

    return pl.pallas_call(
        body,
        out_shape=[jax.ShapeDtypeStruct((N, HID), jnp.float32),
                   jax.ShapeDtypeStruct((NP, W), jnp.float32)],
    )(degp, h1)


def _tc_layer1(parts, g, dinv16, b1):
    def body(p_ref, g_ref, d_ref, b_ref, g2_ref):
        dinv = d_ref[...]
        h = dinv * (p_ref[0, :N, :HID] + p_ref[1, :N, :HID]
                    + g_ref[:N, :HID]) + b_ref[...][None, :]
        h = jnp.maximum(h, 0.0)
        g2_ref[:N, :HID] = dinv * h
        g2_ref[:N, HID:] = jnp.zeros((N, W - HID), jnp.float32)
        g2_ref[N:, :] = jnp.zeros((NP - N, W), jnp.float32)

    return pl.pallas_call(
        body,
        out_shape=jax.ShapeDtypeStruct((NP, W), jnp.float32),
    )(parts, g, dinv16, b1)


def _tc_layer2(parts, g2, dinv16, W2, b2):
    def body(p_ref, g2_ref, d_ref, w_ref, b_ref, o_ref):
        m = d_ref[...] * (p_ref[0, :N, :HID] + p_ref[1, :N, :HID]
                          + g2_ref[:N, :HID])
        o_ref[...] = jnp.dot(m, w_ref[...],
                             preferred_element_type=jnp.float32) \
            + b_ref[...][None, :]

    return pl.pallas_call(
        body,
        out_shape=jax.ShapeDtypeStruct((N, NCL), jnp.float32),
    )(parts, g2, dinv16, W2, b2)


def kernel(x, edge_index, W1, b1, W2, b2):
    idx = edge_index.astype(jnp.int32)
    padrow = jnp.broadcast_to(N + jnp.arange(PAD, dtype=jnp.int32),
                              (NW, PAD))
    src3 = jnp.concatenate([idx[0].reshape(NW, EPW), padrow],
                           axis=1).reshape(NW, NCH, K)
    dst3 = jnp.concatenate([idx[1].reshape(NW, EPW), padrow],
                           axis=1).reshape(NW, NCH, K)
    epack = jnp.stack([src3, dst3], axis=2)

    ones = jnp.ones((K, 1, W), jnp.float32)
    zeros = jnp.zeros((NP, 1, W), jnp.float32)
    degp = _sc_degree(epack, ones, zeros)
    h1 = _tc_matmul1(x, W1)
    dinv16, g = _tc_norm(degp.reshape(NC, NP, W), h1)
    parts1 = _sc_scatter(g.reshape(NP, 1, W), epack, zeros)
    g2 = _tc_layer1(parts1.reshape(NC, NP, W), g, dinv16, b1)
    parts2 = _sc_scatter(g2.reshape(NP, 1, W), epack, zeros)
    return _tc_layer2(parts2.reshape(NC, NP, W), g2, dinv16, W2, b2)
